# baseline (speedup 1.0000x reference)
import functools

import jax
import jax.numpy as jnp
from jax import lax
from jax.experimental import pallas as pl
from jax.experimental.pallas import tpu as pltpu
from jax.experimental.pallas import tpu_sc as plsc

N = 10000
NPAD = 10240
E = 320000
D = 128
H1 = 128
H2 = 64
C = 16

NC, NS, L = 2, 16, 16
NW = NC * NS
EPT = E // NW
CH = 80
NCHUNK = EPT // CH
NPAIR = (NCHUNK - 1) // 2
SLOT = 2 * CH
NSLOTF = EPT // SLOT
NSPAIR = NSLOTF // 2
RPT = NPAD // NS
EP = E + 3 * CH
EROWS = EP // CH
IPT = EPT // CH

_mesh = plsc.VectorSubcoreMesh(core_axis_name="c", subcore_axis_name="s")
_sc_params = pltpu.CompilerParams(use_tc_tiling_on_sc=False,
                                  needs_layout_passes=False)

_f32 = jnp.float32
_i32 = jnp.int32


def _zero_vec(ref, nwords):
    def body(i, _):
        ref[pl.ds(i * L, L)] = jnp.zeros((L,), _f32)
        return 0
    lax.fori_loop(0, nwords // L, body, 0)


@functools.partial(
    pl.kernel,
    out_type=(
        jax.ShapeDtypeStruct((NC, NPAD), _f32),
        jax.ShapeDtypeStruct((NC, NPAD), _f32),
    ),
    mesh=_mesh,
    compiler_params=_sc_params,
    scratch_types=[
        pltpu.VMEM_SHARED((NPAD,), _f32),
        pltpu.VMEM_SHARED((NPAD,), _f32),
        pltpu.VMEM((2, CH), _i32),
        pltpu.VMEM((2, CH), _i32),
        pltpu.VMEM((2, CH), _f32),
        pltpu.VMEM((CH,), _f32),
        pltpu.VMEM((RPT,), _f32),
        pltpu.SemaphoreType.DMA,
        pltpu.SemaphoreType.DMA,
        pltpu.SemaphoreType.DMA,
        pltpu.SemaphoreType.DMA,
    ],
)
def _deg_kernel(tpe_hbm, ei_hbm, ew_hbm, out1_hbm, out2_hbm,
                acc1, acc2, idxa, idxb, ew_v, ones_v, zbuf,
                semI0, semI1, semS0, semS1):
    cid = lax.axis_index("c")
    sid = lax.axis_index("s")
    wid = cid * NS + sid
    _zero_vec(zbuf, RPT)
    for i in range(CH // L):
        ones_v[pl.ds(i * L, L)] = jnp.ones((L,), _f32)
    r0 = sid * RPT
    pltpu.sync_copy(zbuf, acc1.at[pl.ds(r0, RPT)])
    pltpu.sync_copy(zbuf, acc2.at[pl.ds(r0, RPT)])
    plsc.subcore_barrier()
    ebase = wid * EPT
    semI = (semI0, semI1)
    semS = (semS0, semS1)

    def fire_idx(b, c):
        e0 = ebase + c * CH
        pltpu.async_copy(tpe_hbm.at[1, pl.ds(e0, CH)], idxa.at[b], semI[b])
        pltpu.async_copy(ei_hbm.at[1, pl.ds(e0, CH)], idxb.at[b], semI[b])
        pltpu.async_copy(ew_hbm.at[pl.ds(e0, CH)], ew_v.at[b], semI[b])

    def wait_idx(b, c):
        e0 = ebase + c * CH
        pltpu.make_async_copy(
            tpe_hbm.at[1, pl.ds(e0, CH)], idxa.at[b], semI[b]).wait()
        pltpu.make_async_copy(
            ei_hbm.at[1, pl.ds(e0, CH)], idxb.at[b], semI[b]).wait()
        pltpu.make_async_copy(
            ew_hbm.at[pl.ds(e0, CH)], ew_v.at[b], semI[b]).wait()

    def fire_scatter(b):
        pltpu.async_copy(ones_v, acc1.at[idxa.at[b]], semS[b], add=True)
        pltpu.async_copy(ew_v.at[b], acc2.at[idxb.at[b]], semS[b], add=True)

    def wait_scatter(b):
        pltpu.make_async_copy(ones_v, acc1.at[idxa.at[b]], semS[b]).wait()
        pltpu.make_async_copy(ew_v.at[b], acc2.at[idxb.at[b]], semS[b]).wait()

    fire_idx(0, 0)
    fire_idx(1, 1)

    def pair(g, _):
        c0 = 2 * g
        wait_idx(0, c0)
        fire_scatter(0)
        wait_idx(1, c0 + 1)
        fire_scatter(1)
        wait_scatter(0)
        fire_idx(0, c0 + 2)
        wait_scatter(1)
        fire_idx(1, c0 + 3)
        return 0

    lax.fori_loop(0, NPAIR, pair, 0)
    wait_idx(0, NCHUNK - 1)
    fire_scatter(0)
    wait_scatter(0)
    wait_idx(1, NCHUNK)
    plsc.subcore_barrier()
    pltpu.sync_copy(acc1.at[pl.ds(r0, RPT)], out1_hbm.at[cid, pl.ds(r0, RPT)])
    pltpu.sync_copy(acc2.at[pl.ds(r0, RPT)], out2_hbm.at[cid, pl.ds(r0, RPT)])


def _make_agg(F, with_ew):
    scratch = [
        pltpu.VMEM_SHARED((NPAD, F), _f32),
        pltpu.VMEM((2, 2, CH), _i32),
        pltpu.VMEM((2, 2, CH), _i32),
        pltpu.VMEM((SLOT, F), _f32),
        pltpu.VMEM((SLOT, F), _f32),
        pltpu.SemaphoreType.DMA,
        pltpu.SemaphoreType.DMA,
        pltpu.SemaphoreType.DMA,
        pltpu.SemaphoreType.DMA,
        pltpu.SemaphoreType.DMA,
        pltpu.SemaphoreType.DMA,
    ]
    if with_ew:
        scratch += [pltpu.VMEM((SLOT,), _f32), pltpu.VMEM((SLOT,), _f32)]

    @functools.partial(
        pl.kernel,
        out_type=jax.ShapeDtypeStruct((NC, NPAD, F), _f32),
        mesh=_mesh,
        compiler_params=_sc_params,
        scratch_types=scratch,
    )
    def agg(*args):
        if with_ew:
            (g_hbm, eidx_hbm, ew_hbm, out_hbm, acc, I0, I1, rows0, rows1,
             semI0, semI1, semG0, semG1, semS0, semS1, ewb0, ewb1) = args
        else:
            (g_hbm, eidx_hbm, out_hbm, acc, I0, I1, rows0, rows1,
             semI0, semI1, semG0, semG1, semS0, semS1) = args
            ew_hbm = ewb0 = ewb1 = None
        cid = lax.axis_index("c")
        sid = lax.axis_index("s")
        wid = cid * NS + sid

        def zrow(r, _):
            for h in range(F // L):
                rows0[r, pl.ds(h * L, L)] = jnp.zeros((L,), _f32)
            return 0
        lax.fori_loop(0, SLOT, zrow, 0)
        r0 = sid * RPT
        for j in range(RPT // SLOT):
            pltpu.sync_copy(rows0, acc.at[pl.ds(r0 + j * SLOT, SLOT)])
        plsc.subcore_barrier()
        ebase = wid * EPT
        ibase = wid * IPT
        bufs = ((I0, rows0, ewb0, semI0, semG0, semS0),
                (I1, rows1, ewb1, semI1, semG1, semS1))

        def fire_idx(b, s):
            I, _, ewb, semI, _, _ = bufs[b]
            pltpu.async_copy(eidx_hbm.at[:, pl.ds(ibase + 2 * s, 2), :], I, semI)
            if with_ew:
                pltpu.async_copy(ew_hbm.at[pl.ds(ebase + s * SLOT, SLOT)],
                                 ewb, semI)

        def wait_idx(b, s):
            I, _, ewb, semI, _, _ = bufs[b]
            pltpu.make_async_copy(
                eidx_hbm.at[:, pl.ds(ibase + 2 * s, 2), :], I, semI).wait()
            if with_ew:
                pltpu.make_async_copy(
                    ew_hbm.at[pl.ds(ebase + s * SLOT, SLOT)], ewb, semI).wait()

        def fire_gather(b):
            I, rows, _, _, semG, _ = bufs[b]
            for u in range(2):
                pltpu.async_copy(g_hbm.at[I.at[0, u]],
                                 rows.at[pl.ds(u * CH, CH)], semG)

        def wait_gather(b):
            I, rows, _, _, semG, _ = bufs[b]
            for u in range(2):
                pltpu.make_async_copy(g_hbm.at[I.at[0, u]],
                                      rows.at[pl.ds(u * CH, CH)], semG).wait()

        def fire_scatter(b, tail=False):
            I, rows, _, _, _, semS = bufs[b]
            for u in range(1 if tail else 2):
                pltpu.async_copy(rows.at[pl.ds(u * CH, CH)],
                                 acc.at[I.at[1, u]], semS, add=True)

        def wait_scatter(b, tail=False):
            I, rows, _, _, _, semS = bufs[b]
            for u in range(1 if tail else 2):
                pltpu.make_async_copy(rows.at[pl.ds(u * CH, CH)],
                                      acc.at[I.at[1, u]], semS).wait()

        def scale(b):
            if not with_ew:
                return
            _, rows, ewb, _, _, _ = bufs[b]

            @plsc.parallel_loop(0, SLOT, 1)
            def sc_body(e):
                bc = plsc.load_gather(ewb, [jnp.zeros((L,), _i32) + e])
                for h in range(F // L):
                    rows[e, pl.ds(h * L, L)] = rows[e, pl.ds(h * L, L)] * bc

        pltpu.sync_copy(eidx_hbm.at[:, pl.ds(ibase, 2), :], I0)
        if with_ew:
            pltpu.sync_copy(ew_hbm.at[pl.ds(ebase, SLOT)], ewb0)
        fire_gather(0)
        fire_idx(1, 1)

        def pair(g, _):
            s0 = 2 * g
            wait_gather(0)
            scale(0)
            fire_scatter(0)
            wait_idx(1, s0 + 1)
            fire_gather(1)
            wait_scatter(0)
            fire_idx(0, s0 + 2)
            wait_gather(1)
            scale(1)
            fire_scatter(1)
            wait_idx(0, s0 + 2)
            fire_gather(0)
            wait_scatter(1)
            fire_idx(1, s0 + 3)
            return 0

        lax.fori_loop(0, NSPAIR, pair, 0)
        wait_gather(0)
        scale(0)
        fire_scatter(0, tail=True)
        wait_scatter(0, tail=True)
        wait_idx(1, NSLOTF + 1)
        plsc.subcore_barrier()
        pltpu.sync_copy(acc.at[pl.ds(r0, RPT)], out_hbm.at[cid, pl.ds(r0, RPT)])

    return agg


_agg128 = _make_agg(D, False)
_agg64 = _make_agg(H2, False)
_agg32ew = _make_agg(2 * C, True)


@functools.partial(
    pl.kernel,
    out_type=jax.ShapeDtypeStruct((E,), _f32),
    mesh=_mesh,
    compiler_params=_sc_params,
    scratch_types=[
        pltpu.VMEM((2, 2, CH), _i32),
        pltpu.VMEM((2, 2, CH), _i32),
        pltpu.VMEM((SLOT, H2), _f32),
        pltpu.VMEM((SLOT, H2), _f32),
        pltpu.VMEM((SLOT, H2), _f32),
        pltpu.VMEM((SLOT, H2), _f32),
        pltpu.VMEM((SLOT,), _f32),
        pltpu.VMEM((SLOT,), _f32),
        pltpu.VMEM((SLOT // L, L, L), _f32),
        pltpu.SemaphoreType.DMA,
        pltpu.SemaphoreType.DMA,
        pltpu.SemaphoreType.DMA,
        pltpu.SemaphoreType.DMA,
        pltpu.SemaphoreType.DMA,
        pltpu.SemaphoreType.DMA,
    ],
)
def _dot_kernel(hf_hbm, tei_hbm, out_hbm,
                I0, I1, a0, b0, a1, b1, res0, res1, tmp,
                semI0, semI1, semG0, semG1, semR0, semR1):
    cid = lax.axis_index("c")
    sid = lax.axis_index("s")
    wid = cid * NS + sid
    ebase = wid * EPT
    ibase = wid * IPT
    iota = lax.iota(_i32, L)
    bufs = ((I0, a0, b0, res0, semI0, semG0, semR0),
            (I1, a1, b1, res1, semI1, semG1, semR1))

    def fire_idx(b, s):
        I, _, _, _, semI, _, _ = bufs[b]
        pltpu.async_copy(tei_hbm.at[:, pl.ds(ibase + 2 * s, 2), :], I, semI)

    def wait_idx(b, s):
        I, _, _, _, semI, _, _ = bufs[b]
        pltpu.make_async_copy(
            tei_hbm.at[:, pl.ds(ibase + 2 * s, 2), :], I, semI).wait()

    def fire_gather(b):
        I, av, bv, _, _, semG, _ = bufs[b]
        for u in range(2):
            pltpu.async_copy(hf_hbm.at[I.at[0, u]],
                             av.at[pl.ds(u * CH, CH)], semG)
            pltpu.async_copy(hf_hbm.at[I.at[1, u]],
                             bv.at[pl.ds(u * CH, CH)], semG)

    def wait_gather(b):
        I, av, bv, _, _, semG, _ = bufs[b]
        for u in range(2):
            pltpu.make_async_copy(hf_hbm.at[I.at[0, u]],
                                  av.at[pl.ds(u * CH, CH)], semG).wait()
            pltpu.make_async_copy(hf_hbm.at[I.at[1, u]],
                                  bv.at[pl.ds(u * CH, CH)], semG).wait()

    def fire_res(b, s, tail=False):
        _, _, _, res, _, _, semR = bufs[b]
        n = CH if tail else SLOT
        pltpu.async_copy(res.at[pl.ds(0, n)],
                         out_hbm.at[pl.ds(ebase + s * SLOT, n)], semR)

    def wait_res(b, s, tail=False):
        _, _, _, res, _, _, semR = bufs[b]
        n = CH if tail else SLOT
        pltpu.make_async_copy(res.at[pl.ds(0, n)],
                              out_hbm.at[pl.ds(ebase + s * SLOT, n)],
                              semR).wait()

    def compute(b):
        _, av, bv, res, _, _, _ = bufs[b]

        @plsc.parallel_loop(0, SLOT // L, 1, unroll=2)
        def grp(g):
            tg = tmp.at[g]
            for l in range(L):
                e = g * L + l
                m = av[e, pl.ds(0, L)] * bv[e, pl.ds(0, L)]
                for h in range(1, H2 // L):
                    m = m + av[e, pl.ds(h * L, L)] * bv[e, pl.ds(h * L, L)]
                tg[l, pl.ds(0, L)] = m
            rsum = jnp.zeros((L,), _f32)
            for cc in range(L):
                rsum = rsum + plsc.load_gather(tg, [iota, jnp.full((L,), cc, _i32)])
            res[pl.ds(g * L, L)] = rsum

    pltpu.sync_copy(tei_hbm.at[:, pl.ds(ibase, 2), :], I0)
    fire_gather(0)
    fire_idx(1, 1)
    fire_res(0, 0)
    fire_res(1, 1)

    def pair(g, _):
        s0 = 2 * g
        wait_gather(0)
        wait_res(0, s0)
        compute(0)
        fire_res(0, s0)
        wait_idx(1, s0 + 1)
        fire_gather(1)
        fire_idx(0, s0 + 2)
        wait_gather(1)
        wait_res(1, s0 + 1)
        compute(1)
        fire_res(1, s0 + 1)
        wait_idx(0, s0 + 2)
        fire_gather(0)
        fire_idx(1, s0 + 3)
        return 0

    lax.fori_loop(0, NSPAIR, pair, 0)
    wait_gather(0)
    wait_res(0, NSLOTF - 2)
    compute(0)
    fire_res(0, NSLOTF, tail=True)
    wait_res(0, NSLOTF, tail=True)
    wait_res(1, NSLOTF - 1)
    wait_idx(1, NSLOTF + 1)


_BLK = 1000
_G = N // _BLK


def _m1_body(x_ref, w_ref, t1_ref, t2_ref, g1_ref, dinv_ref):
    t1 = t1_ref[...]
    t2 = t2_ref[...]
    d1 = lax.rsqrt(1.0 + t1[:, 0:1] + t1[:, 1:2])
    d2 = lax.rsqrt(1.0 + t2[:, 0:1] + t2[:, 1:2])
    h = jnp.dot(x_ref[...], w_ref[...], preferred_element_type=_f32)
    g1_ref[...] = d1 * h
    dinv_ref[...] = jnp.concatenate([d1, d2], axis=1)


def _m2_body(p_ref, g1_ref, dinv_ref, b1_ref, w2_ref, g2_ref):
    d1 = dinv_ref[:, 0:1]
    z = jnp.maximum(d1 * (p_ref[0] + p_ref[1] + g1_ref[...]) + b1_ref[...], 0.0)
    g2_ref[...] = d1 * jnp.dot(z, w2_ref[...], preferred_element_type=_f32)


def _m3_body(p_ref, g2_ref, dinv_ref, b2_ref, wc_ref, hf_ref, g3_ref):
    d1 = dinv_ref[:, 0:1]
    d2 = dinv_ref[:, 1:2]
    hf = d1 * (p_ref[0] + p_ref[1] + g2_ref[...]) + b2_ref[...]
    hf_ref[...] = hf
    g3_ref[...] = d2 * jnp.dot(hf, wc_ref[...], preferred_element_type=_f32)


def _m4_body(p_ref, g3_ref, dinv_ref, bc_ref, attr_ref, att_ref):
    d2 = dinv_ref[:, 1:2]
    o = d2 * (p_ref[0] + p_ref[1] + g3_ref[...]) + bc_ref[...]

    def ls(v):
        m = jnp.max(v, axis=1, keepdims=True)
        return v - m - jnp.log(jnp.sum(jnp.exp(v - m), axis=1, keepdims=True))

    attr_ref[...] = ls(o[:, :C])
    att_ref[...] = ls(o[:, C:])


def _row_spec(f):
    return pl.BlockSpec((_BLK, f), lambda i: (i, 0))


def _part_spec(f):
    return pl.BlockSpec((NC, _BLK, f), lambda i: (0, i, 0))


def _full_spec(a, b):
    return pl.BlockSpec((a, b), lambda i: (0, 0))


def _m1(x, W1, t1, t2):
    return pl.pallas_call(
        _m1_body,
        grid=(_G,),
        in_specs=[_row_spec(D), _full_spec(D, H1), _row_spec(2), _row_spec(2)],
        out_specs=[_row_spec(H1), _row_spec(2)],
        out_shape=[jax.ShapeDtypeStruct((N, H1), _f32),
                   jax.ShapeDtypeStruct((N, 2), _f32)],
    )(x, W1, t1, t2)


def _m2(P1, g1, dinv, b1, W2):
    return pl.pallas_call(
        _m2_body,
        grid=(_G,),
        in_specs=[_part_spec(H1), _row_spec(H1), _row_spec(2),
                  _full_spec(1, H1), _full_spec(H1, H2)],
        out_specs=_row_spec(H2),
        out_shape=jax.ShapeDtypeStruct((N, H2), _f32),
    )(P1, g1, dinv, b1, W2)


def _m3(P2, g2, dinv, b2, Wc):
    return pl.pallas_call(
        _m3_body,
        grid=(_G,),
        in_specs=[_part_spec(H2), _row_spec(H2), _row_spec(2),
                  _full_spec(1, H2), _full_spec(H2, 2 * C)],
        out_specs=[_row_spec(H2), _row_spec(2 * C)],
        out_shape=[jax.ShapeDtypeStruct((N, H2), _f32),
                   jax.ShapeDtypeStruct((N, 2 * C), _f32)],
    )(P2, g2, dinv, b2, Wc)


def _m4(P3, g3, dinv, bc):
    return pl.pallas_call(
        _m4_body,
        grid=(_G,),
        in_specs=[_part_spec(2 * C), _row_spec(2 * C), _row_spec(2),
                  _full_spec(1, 2 * C)],
        out_specs=[_row_spec(C), _row_spec(C)],
        out_shape=[jax.ShapeDtypeStruct((N, C), _f32),
                   jax.ShapeDtypeStruct((N, C), _f32)],
    )(P3, g3, dinv, bc)


def kernel(x, edge_weight, W1, b1, W2, b2, Wattr, battr, Watt, batt,
           train_pos_edge_index, edge_index, pos_edge_index, neg_edge_index):
    tpe = jnp.pad(train_pos_edge_index, ((0, 0), (0, EP - E)))
    ei = jnp.pad(edge_index, ((0, 0), (0, EP - E)))
    ewp = jnp.pad(edge_weight, (0, EP - E))
    tpe3 = tpe.reshape(2, EROWS, CH)
    ei3 = ei.reshape(2, EROWS, CH)

    degp1, degp2 = _deg_kernel(tpe, ei, ewp)
    t1 = degp1[:, :N].T
    t2 = degp2[:, :N].T

    g1, dinv = _m1(x, W1, t1, t2)
    P1 = _agg128(g1, tpe3)[:, :N, :]
    g2 = _m2(P1, g1, dinv, b1.reshape(1, H1), W2)
    P2 = _agg64(g2, tpe3)[:, :N, :]
    Wc = jnp.concatenate([Wattr, Watt], axis=1)
    bc = jnp.concatenate([battr, batt]).reshape(1, 2 * C)
    hf, g3 = _m3(P2, g2, dinv, b2.reshape(1, H2), Wc)
    P3 = _agg32ew(g3, ei3, ewp)[:, :N, :]
    attr, att = _m4(P3, g3, dinv, bc)

    tei3 = jnp.pad(jnp.concatenate([pos_edge_index, neg_edge_index], axis=-1),
                   ((0, 0), (0, EP - E))).reshape(2, EROWS, CH)
    res = _dot_kernel(hf, tei3)
    return (res, attr, att)


# speedup vs baseline: 1.0050x; 1.0050x over previous
import functools

import jax
import jax.numpy as jnp
from jax import lax
from jax.experimental import pallas as pl
from jax.experimental.pallas import tpu as pltpu
from jax.experimental.pallas import tpu_sc as plsc

N = 10000
NPAD = 10240
E = 320000
D = 128
H1 = 128
H2 = 64
C = 16

NC, NS, L = 2, 16, 16
NW = NC * NS
EPT = E // NW
CH = 80
NCHUNK = EPT // CH
NPAIR = (NCHUNK - 1) // 2
SLOT = 2 * CH
NSLOTF = EPT // SLOT
NSPAIR = NSLOTF // 2
RPT = NPAD // NS
EP = E + 3 * CH
EROWS = EP // CH
IPT = EPT // CH

_mesh = plsc.VectorSubcoreMesh(core_axis_name="c", subcore_axis_name="s")
_sc_params = pltpu.CompilerParams(use_tc_tiling_on_sc=False,
                                  needs_layout_passes=False)

_f32 = jnp.float32
_i32 = jnp.int32


def _zero_vec(ref, nwords):
    def body(i, _):
        ref[pl.ds(i * L, L)] = jnp.zeros((L,), _f32)
        return 0
    lax.fori_loop(0, nwords // L, body, 0)


@functools.partial(
    pl.kernel,
    out_type=(
        jax.ShapeDtypeStruct((NC, NPAD), _f32),
        jax.ShapeDtypeStruct((NC, NPAD), _f32),
    ),
    mesh=_mesh,
    compiler_params=_sc_params,
    scratch_types=[
        pltpu.VMEM_SHARED((NPAD,), _f32),
        pltpu.VMEM_SHARED((NPAD,), _f32),
        pltpu.VMEM((2, CH), _i32),
        pltpu.VMEM((2, CH), _i32),
        pltpu.VMEM((2, CH), _f32),
        pltpu.VMEM((CH,), _f32),
        pltpu.VMEM((RPT,), _f32),
        pltpu.SemaphoreType.DMA,
        pltpu.SemaphoreType.DMA,
        pltpu.SemaphoreType.DMA,
        pltpu.SemaphoreType.DMA,
    ],
)
def _deg_kernel(tpe_hbm, ei_hbm, ew_hbm, out1_hbm, out2_hbm,
                acc1, acc2, idxa, idxb, ew_v, ones_v, zbuf,
                semI0, semI1, semS0, semS1):
    cid = lax.axis_index("c")
    sid = lax.axis_index("s")
    wid = cid * NS + sid
    _zero_vec(zbuf, RPT)
    for i in range(CH // L):
        ones_v[pl.ds(i * L, L)] = jnp.ones((L,), _f32)
    r0 = sid * RPT
    pltpu.sync_copy(zbuf, acc1.at[pl.ds(r0, RPT)])
    pltpu.sync_copy(zbuf, acc2.at[pl.ds(r0, RPT)])
    plsc.subcore_barrier()
    ebase = wid * EPT
    semI = (semI0, semI1)
    semS = (semS0, semS1)

    def fire_idx(b, c):
        e0 = ebase + c * CH
        pltpu.async_copy(tpe_hbm.at[1, pl.ds(e0, CH)], idxa.at[b], semI[b])
        pltpu.async_copy(ei_hbm.at[1, pl.ds(e0, CH)], idxb.at[b], semI[b])
        pltpu.async_copy(ew_hbm.at[pl.ds(e0, CH)], ew_v.at[b], semI[b])

    def wait_idx(b, c):
        e0 = ebase + c * CH
        pltpu.make_async_copy(
            tpe_hbm.at[1, pl.ds(e0, CH)], idxa.at[b], semI[b]).wait()
        pltpu.make_async_copy(
            ei_hbm.at[1, pl.ds(e0, CH)], idxb.at[b], semI[b]).wait()
        pltpu.make_async_copy(
            ew_hbm.at[pl.ds(e0, CH)], ew_v.at[b], semI[b]).wait()

    def fire_scatter(b):
        pltpu.async_copy(ones_v, acc1.at[idxa.at[b]], semS[b], add=True)
        pltpu.async_copy(ew_v.at[b], acc2.at[idxb.at[b]], semS[b], add=True)

    def wait_scatter(b):
        pltpu.make_async_copy(ones_v, acc1.at[idxa.at[b]], semS[b]).wait()
        pltpu.make_async_copy(ew_v.at[b], acc2.at[idxb.at[b]], semS[b]).wait()

    fire_idx(0, 0)
    fire_idx(1, 1)

    def pair(g, _):
        c0 = 2 * g
        wait_idx(0, c0)
        fire_scatter(0)
        wait_idx(1, c0 + 1)
        fire_scatter(1)
        wait_scatter(0)
        fire_idx(0, c0 + 2)
        wait_scatter(1)
        fire_idx(1, c0 + 3)
        return 0

    lax.fori_loop(0, NPAIR, pair, 0)
    wait_idx(0, NCHUNK - 1)
    fire_scatter(0)
    wait_scatter(0)
    wait_idx(1, NCHUNK)
    plsc.subcore_barrier()
    pltpu.sync_copy(acc1.at[pl.ds(r0, RPT)], out1_hbm.at[cid, pl.ds(r0, RPT)])
    pltpu.sync_copy(acc2.at[pl.ds(r0, RPT)], out2_hbm.at[cid, pl.ds(r0, RPT)])


def _make_agg(F, with_ew):
    scratch = [
        pltpu.VMEM_SHARED((NPAD, F), _f32),
        pltpu.VMEM((2, 2, CH), _i32),
        pltpu.VMEM((2, 2, CH), _i32),
        pltpu.VMEM((SLOT, F), _f32),
        pltpu.VMEM((SLOT, F), _f32),
        pltpu.SemaphoreType.DMA,
        pltpu.SemaphoreType.DMA,
        pltpu.SemaphoreType.DMA,
        pltpu.SemaphoreType.DMA,
        pltpu.SemaphoreType.DMA,
        pltpu.SemaphoreType.DMA,
    ]
    if with_ew:
        scratch += [pltpu.VMEM((SLOT,), _f32), pltpu.VMEM((SLOT,), _f32)]

    @functools.partial(
        pl.kernel,
        out_type=jax.ShapeDtypeStruct((NC, NPAD, F), _f32),
        mesh=_mesh,
        compiler_params=_sc_params,
        scratch_types=scratch,
    )
    def agg(*args):
        if with_ew:
            (g_hbm, eidx_hbm, ew_hbm, out_hbm, acc, I0, I1, rows0, rows1,
             semI0, semI1, semG0, semG1, semS0, semS1, ewb0, ewb1) = args
        else:
            (g_hbm, eidx_hbm, out_hbm, acc, I0, I1, rows0, rows1,
             semI0, semI1, semG0, semG1, semS0, semS1) = args
            ew_hbm = ewb0 = ewb1 = None
        cid = lax.axis_index("c")
        sid = lax.axis_index("s")
        wid = cid * NS + sid

        def zrow(r, _):
            for h in range(F // L):
                rows0[r, pl.ds(h * L, L)] = jnp.zeros((L,), _f32)
            return 0
        lax.fori_loop(0, SLOT, zrow, 0)
        r0 = sid * RPT
        for j in range(RPT // SLOT):
            pltpu.sync_copy(rows0, acc.at[pl.ds(r0 + j * SLOT, SLOT)])
        plsc.subcore_barrier()
        ebase = wid * EPT
        ibase = wid * IPT
        bufs = ((I0, rows0, ewb0, semI0, semG0, semS0),
                (I1, rows1, ewb1, semI1, semG1, semS1))

        def fire_idx(b, s):
            I, _, ewb, semI, _, _ = bufs[b]
            pltpu.async_copy(eidx_hbm.at[:, pl.ds(ibase + 2 * s, 2), :], I, semI)
            if with_ew:
                pltpu.async_copy(ew_hbm.at[pl.ds(ebase + s * SLOT, SLOT)],
                                 ewb, semI)

        def wait_idx(b, s):
            I, _, ewb, semI, _, _ = bufs[b]
            pltpu.make_async_copy(
                eidx_hbm.at[:, pl.ds(ibase + 2 * s, 2), :], I, semI).wait()
            if with_ew:
                pltpu.make_async_copy(
                    ew_hbm.at[pl.ds(ebase + s * SLOT, SLOT)], ewb, semI).wait()

        def fire_gather(b):
            I, rows, _, _, semG, _ = bufs[b]
            for u in range(2):
                pltpu.async_copy(g_hbm.at[I.at[0, u]],
                                 rows.at[pl.ds(u * CH, CH)], semG)

        def wait_gather(b):
            I, rows, _, _, semG, _ = bufs[b]
            for u in range(2):
                pltpu.make_async_copy(g_hbm.at[I.at[0, u]],
                                      rows.at[pl.ds(u * CH, CH)], semG).wait()

        def fire_scatter(b, tail=False):
            I, rows, _, _, _, semS = bufs[b]
            for u in range(1 if tail else 2):
                pltpu.async_copy(rows.at[pl.ds(u * CH, CH)],
                                 acc.at[I.at[1, u]], semS, add=True)

        def wait_scatter(b, tail=False):
            I, rows, _, _, _, semS = bufs[b]
            for u in range(1 if tail else 2):
                pltpu.make_async_copy(rows.at[pl.ds(u * CH, CH)],
                                      acc.at[I.at[1, u]], semS).wait()

        def scale(b):
            if not with_ew:
                return
            _, rows, ewb, _, _, _ = bufs[b]

            @plsc.parallel_loop(0, SLOT, 1)
            def sc_body(e):
                bc = plsc.load_gather(ewb, [jnp.zeros((L,), _i32) + e])
                for h in range(F // L):
                    rows[e, pl.ds(h * L, L)] = rows[e, pl.ds(h * L, L)] * bc

        pltpu.sync_copy(eidx_hbm.at[:, pl.ds(ibase, 2), :], I0)
        if with_ew:
            pltpu.sync_copy(ew_hbm.at[pl.ds(ebase, SLOT)], ewb0)
        fire_gather(0)
        fire_idx(1, 1)

        def pair(g, _):
            s0 = 2 * g
            wait_gather(0)
            scale(0)
            fire_scatter(0)
            wait_idx(1, s0 + 1)
            fire_gather(1)
            wait_scatter(0)
            fire_idx(0, s0 + 2)
            wait_gather(1)
            scale(1)
            fire_scatter(1)
            wait_idx(0, s0 + 2)
            fire_gather(0)
            wait_scatter(1)
            fire_idx(1, s0 + 3)
            return 0

        lax.fori_loop(0, NSPAIR, pair, 0)
        wait_gather(0)
        scale(0)
        fire_scatter(0, tail=True)
        wait_scatter(0, tail=True)
        wait_idx(1, NSLOTF + 1)
        plsc.subcore_barrier()
        pltpu.sync_copy(acc.at[pl.ds(r0, RPT)], out_hbm.at[cid, pl.ds(r0, RPT)])

    return agg


_agg128 = _make_agg(D, False)
_agg64 = _make_agg(H2, False)
_agg32ew = _make_agg(2 * C, True)


@functools.partial(
    pl.kernel,
    out_type=jax.ShapeDtypeStruct((E,), _f32),
    mesh=_mesh,
    compiler_params=_sc_params,
    scratch_types=[
        pltpu.VMEM((2, 2, CH), _i32),
        pltpu.VMEM((2, 2, CH), _i32),
        pltpu.VMEM((SLOT, H2), _f32),
        pltpu.VMEM((SLOT, H2), _f32),
        pltpu.VMEM((SLOT, H2), _f32),
        pltpu.VMEM((SLOT, H2), _f32),
        pltpu.VMEM((SLOT,), _f32),
        pltpu.VMEM((SLOT,), _f32),
        pltpu.VMEM((SLOT // L, L, L), _f32),
        pltpu.SemaphoreType.DMA,
        pltpu.SemaphoreType.DMA,
        pltpu.SemaphoreType.DMA,
        pltpu.SemaphoreType.DMA,
        pltpu.SemaphoreType.DMA,
        pltpu.SemaphoreType.DMA,
    ],
)
def _dot_kernel(hf_hbm, tei_hbm, out_hbm,
                I0, I1, a0, b0, a1, b1, res0, res1, tmp,
                semI0, semI1, semG0, semG1, semR0, semR1):
    cid = lax.axis_index("c")
    sid = lax.axis_index("s")
    wid = cid * NS + sid
    ebase = wid * EPT
    ibase = wid * IPT
    iota = lax.iota(_i32, L)
    bufs = ((I0, a0, b0, res0, semI0, semG0, semR0),
            (I1, a1, b1, res1, semI1, semG1, semR1))

    def fire_idx(b, s):
        I, _, _, _, semI, _, _ = bufs[b]
        pltpu.async_copy(tei_hbm.at[:, pl.ds(ibase + 2 * s, 2), :], I, semI)

    def wait_idx(b, s):
        I, _, _, _, semI, _, _ = bufs[b]
        pltpu.make_async_copy(
            tei_hbm.at[:, pl.ds(ibase + 2 * s, 2), :], I, semI).wait()

    def fire_gather(b):
        I, av, bv, _, _, semG, _ = bufs[b]
        for u in range(2):
            pltpu.async_copy(hf_hbm.at[I.at[0, u]],
                             av.at[pl.ds(u * CH, CH)], semG)
            pltpu.async_copy(hf_hbm.at[I.at[1, u]],
                             bv.at[pl.ds(u * CH, CH)], semG)

    def wait_gather(b):
        I, av, bv, _, _, semG, _ = bufs[b]
        for u in range(2):
            pltpu.make_async_copy(hf_hbm.at[I.at[0, u]],
                                  av.at[pl.ds(u * CH, CH)], semG).wait()
            pltpu.make_async_copy(hf_hbm.at[I.at[1, u]],
                                  bv.at[pl.ds(u * CH, CH)], semG).wait()

    def fire_res(b, s, tail=False):
        _, _, _, res, _, _, semR = bufs[b]
        n = CH if tail else SLOT
        pltpu.async_copy(res.at[pl.ds(0, n)],
                         out_hbm.at[pl.ds(ebase + s * SLOT, n)], semR)

    def wait_res(b, s, tail=False):
        _, _, _, res, _, _, semR = bufs[b]
        n = CH if tail else SLOT
        pltpu.make_async_copy(res.at[pl.ds(0, n)],
                              out_hbm.at[pl.ds(ebase + s * SLOT, n)],
                              semR).wait()

    def compute(b):
        _, av, bv, res, _, _, _ = bufs[b]

        @plsc.parallel_loop(0, SLOT // L, 1)
        def grp(g):
            tg = tmp.at[g]
            for l in range(L):
                e = g * L + l
                m = av[e, pl.ds(0, L)] * bv[e, pl.ds(0, L)]
                for h in range(1, H2 // L):
                    m = m + av[e, pl.ds(h * L, L)] * bv[e, pl.ds(h * L, L)]
                tg[l, pl.ds(0, L)] = m
            rsum = jnp.zeros((L,), _f32)
            for cc in range(L):
                rsum = rsum + plsc.load_gather(tg, [iota, jnp.full((L,), cc, _i32)])
            res[pl.ds(g * L, L)] = rsum

    pltpu.sync_copy(tei_hbm.at[:, pl.ds(ibase, 2), :], I0)
    fire_gather(0)
    fire_idx(1, 1)
    fire_res(0, 0)
    fire_res(1, 1)

    def pair(g, _):
        s0 = 2 * g
        wait_gather(0)
        wait_res(0, s0)
        compute(0)
        fire_res(0, s0)
        wait_idx(1, s0 + 1)
        fire_gather(1)
        fire_idx(0, s0 + 2)
        wait_gather(1)
        wait_res(1, s0 + 1)
        compute(1)
        fire_res(1, s0 + 1)
        wait_idx(0, s0 + 2)
        fire_gather(0)
        fire_idx(1, s0 + 3)
        return 0

    lax.fori_loop(0, NSPAIR, pair, 0)
    wait_gather(0)
    wait_res(0, NSLOTF - 2)
    compute(0)
    fire_res(0, NSLOTF, tail=True)
    wait_res(0, NSLOTF, tail=True)
    wait_res(1, NSLOTF - 1)
    wait_idx(1, NSLOTF + 1)


_BLK = 1000
_G = N // _BLK


def _m1_body(x_ref, w_ref, t1_ref, t2_ref, g1_ref, dinv_ref):
    t1 = t1_ref[...]
    t2 = t2_ref[...]
    d1 = lax.rsqrt(1.0 + t1[:, 0:1] + t1[:, 1:2])
    d2 = lax.rsqrt(1.0 + t2[:, 0:1] + t2[:, 1:2])
    h = jnp.dot(x_ref[...], w_ref[...], preferred_element_type=_f32)
    g1_ref[...] = d1 * h
    dinv_ref[...] = jnp.concatenate([d1, d2], axis=1)


def _m2_body(p_ref, g1_ref, dinv_ref, b1_ref, w2_ref, g2_ref):
    d1 = dinv_ref[:, 0:1]
    z = jnp.maximum(d1 * (p_ref[0] + p_ref[1] + g1_ref[...]) + b1_ref[...], 0.0)
    g2_ref[...] = d1 * jnp.dot(z, w2_ref[...], preferred_element_type=_f32)


def _m3_body(p_ref, g2_ref, dinv_ref, b2_ref, wc_ref, hf_ref, g3_ref):
    d1 = dinv_ref[:, 0:1]
    d2 = dinv_ref[:, 1:2]
    hf = d1 * (p_ref[0] + p_ref[1] + g2_ref[...]) + b2_ref[...]
    hf_ref[...] = hf
    g3_ref[...] = d2 * jnp.dot(hf, wc_ref[...], preferred_element_type=_f32)


def _m4_body(p_ref, g3_ref, dinv_ref, bc_ref, attr_ref, att_ref):
    d2 = dinv_ref[:, 1:2]
    o = d2 * (p_ref[0] + p_ref[1] + g3_ref[...]) + bc_ref[...]

    def ls(v):
        m = jnp.max(v, axis=1, keepdims=True)
        return v - m - jnp.log(jnp.sum(jnp.exp(v - m), axis=1, keepdims=True))

    attr_ref[...] = ls(o[:, :C])
    att_ref[...] = ls(o[:, C:])


def _row_spec(f):
    return pl.BlockSpec((_BLK, f), lambda i: (i, 0))


def _part_spec(f):
    return pl.BlockSpec((NC, _BLK, f), lambda i: (0, i, 0))


def _full_spec(a, b):
    return pl.BlockSpec((a, b), lambda i: (0, 0))


def _m1(x, W1, t1, t2):
    return pl.pallas_call(
        _m1_body,
        grid=(_G,),
        in_specs=[_row_spec(D), _full_spec(D, H1), _row_spec(2), _row_spec(2)],
        out_specs=[_row_spec(H1), _row_spec(2)],
        out_shape=[jax.ShapeDtypeStruct((N, H1), _f32),
                   jax.ShapeDtypeStruct((N, 2), _f32)],
    )(x, W1, t1, t2)


def _m2(P1, g1, dinv, b1, W2):
    return pl.pallas_call(
        _m2_body,
        grid=(_G,),
        in_specs=[_part_spec(H1), _row_spec(H1), _row_spec(2),
                  _full_spec(1, H1), _full_spec(H1, H2)],
        out_specs=_row_spec(H2),
        out_shape=jax.ShapeDtypeStruct((N, H2), _f32),
    )(P1, g1, dinv, b1, W2)


def _m3(P2, g2, dinv, b2, Wc):
    return pl.pallas_call(
        _m3_body,
        grid=(_G,),
        in_specs=[_part_spec(H2), _row_spec(H2), _row_spec(2),
                  _full_spec(1, H2), _full_spec(H2, 2 * C)],
        out_specs=[_row_spec(H2), _row_spec(2 * C)],
        out_shape=[jax.ShapeDtypeStruct((N, H2), _f32),
                   jax.ShapeDtypeStruct((N, 2 * C), _f32)],
    )(P2, g2, dinv, b2, Wc)


def _m4(P3, g3, dinv, bc):
    return pl.pallas_call(
        _m4_body,
        grid=(_G,),
        in_specs=[_part_spec(2 * C), _row_spec(2 * C), _row_spec(2),
                  _full_spec(1, 2 * C)],
        out_specs=[_row_spec(C), _row_spec(C)],
        out_shape=[jax.ShapeDtypeStruct((N, C), _f32),
                   jax.ShapeDtypeStruct((N, C), _f32)],
    )(P3, g3, dinv, bc)


def kernel(x, edge_weight, W1, b1, W2, b2, Wattr, battr, Watt, batt,
           train_pos_edge_index, edge_index, pos_edge_index, neg_edge_index):
    tpe = jnp.pad(train_pos_edge_index, ((0, 0), (0, EP - E)))
    ei = jnp.pad(edge_index, ((0, 0), (0, EP - E)))
    ewp = jnp.pad(edge_weight, (0, EP - E))
    tpe3 = tpe.reshape(2, EROWS, CH)
    ei3 = ei.reshape(2, EROWS, CH)

    degp1, degp2 = _deg_kernel(tpe, ei, ewp)
    t1 = degp1[:, :N].T
    t2 = degp2[:, :N].T

    g1, dinv = _m1(x, W1, t1, t2)
    P1 = _agg128(g1, tpe3)[:, :N, :]
    g2 = _m2(P1, g1, dinv, b1.reshape(1, H1), W2)
    P2 = _agg64(g2, tpe3)[:, :N, :]
    Wc = jnp.concatenate([Wattr, Watt], axis=1)
    bc = jnp.concatenate([battr, batt]).reshape(1, 2 * C)
    hf, g3 = _m3(P2, g2, dinv, b2.reshape(1, H2), Wc)
    P3 = _agg32ew(g3, ei3, ewp)[:, :N, :]
    attr, att = _m4(P3, g3, dinv, bc)

    tei3 = jnp.pad(jnp.concatenate([pos_edge_index, neg_edge_index], axis=-1),
                   ((0, 0), (0, EP - E))).reshape(2, EROWS, CH)
    res = _dot_kernel(hf, tei3)
    return (res, attr, att)
